# Initial kernel scaffold; baseline (speedup 1.0000x reference)
import functools

import jax
import jax.numpy as jnp
from jax import lax
from jax.experimental import pallas as pl
from jax.experimental.pallas import tpu as pltpu
from jax.experimental.pallas import tpu_sc as plsc

E = 160000
N = 10000
HIDDEN = 256


BLK_E = 640


def _silu(x):
    return x * (1.0 / (1.0 + jnp.exp(-x)))


def _mlp_body(ff_ref, ev_ref, w0_ref, w1_ref, w2_ref, w3_ref, w4_ref,
              b_ref, out_ref):
    h = ff_ref[...]
    h = _silu(jax.lax.dot_general(h, w0_ref[...], (((1,), (0,)), ((), ())),
                                  preferred_element_type=jnp.float32)
              + b_ref[0, :])
    h = _silu(jax.lax.dot_general(h, w1_ref[...], (((1,), (0,)), ((), ())),
                                  preferred_element_type=jnp.float32)
              + b_ref[1, :])
    h = _silu(jax.lax.dot_general(h, w2_ref[...], (((1,), (0,)), ((), ())),
                                  preferred_element_type=jnp.float32)
              + b_ref[2, :])
    h = _silu(jax.lax.dot_general(h, w3_ref[...], (((1,), (0,)), ((), ())),
                                  preferred_element_type=jnp.float32)
              + b_ref[3, :])
    scale = jnp.sum(h * w4_ref[...], axis=1, keepdims=True) + b_ref[4, 0]
    out_ref[...] = scale * ev_ref[...]


def _mlp_stage(ff, ev4, W0, W1, W2, W3, w4row, ballb4):
    grid = (E // BLK_E,)
    return pl.pallas_call(
        _mlp_body,
        grid=grid,
        in_specs=[
            pl.BlockSpec((BLK_E, HIDDEN), lambda i: (i, 0)),
            pl.BlockSpec((BLK_E, 4), lambda i: (i, 0)),
            pl.BlockSpec((HIDDEN, HIDDEN), lambda i: (0, 0)),
            pl.BlockSpec((HIDDEN, HIDDEN), lambda i: (0, 0)),
            pl.BlockSpec((HIDDEN, HIDDEN), lambda i: (0, 0)),
            pl.BlockSpec((HIDDEN, HIDDEN), lambda i: (0, 0)),
            pl.BlockSpec((1, HIDDEN), lambda i: (0, 0)),
            pl.BlockSpec((8, HIDDEN), lambda i: (0, 0)),
        ],
        out_specs=pl.BlockSpec((BLK_E, 4), lambda i: (i, 0)),
        out_shape=jax.ShapeDtypeStruct((E, 4), jnp.float32),
    )(ff, ev4, W0, W1, W2, W3, w4row, ballb4)


NC, NS = 2, 16
NW = NC * NS
EPW = E // NW
SCHUNK = 100
NSUB = EPW // SCHUNK
ZROWS = 640


def _scatter_body(val_hbm, idx_hbm, out_hbm, idx_v, val_v, zero_v, acc_sh):
    cid = lax.axis_index("c")
    sid = lax.axis_index("s")
    wid = sid * NC + cid

    def _z(i, _):
        zero_v[pl.ds(i * 16, 16)] = jnp.zeros((16,), jnp.float32)
        return 0
    lax.fori_loop(0, (ZROWS * 4) // 16, _z, 0)

    zbase = sid * ZROWS
    last = N - 15 * ZROWS

    @pl.when(sid < NS - 1)
    def _():
        pltpu.sync_copy(zero_v.at[pl.ds(0, ZROWS * 4)],
                        acc_sh.at[pl.ds(zbase * 4, ZROWS * 4)])

    @pl.when(sid == NS - 1)
    def _():
        pltpu.sync_copy(zero_v.at[pl.ds(0, last * 4)],
                        acc_sh.at[pl.ds(15 * ZROWS * 4, last * 4)])

    pltpu.sync_copy(idx_hbm.at[wid], idx_v)
    pltpu.sync_copy(val_hbm.at[wid], val_v)

    plsc.subcore_barrier()

    def _scat(j, _):
        pltpu.sync_copy(val_v.at[pl.ds(j * SCHUNK, SCHUNK)],
                        acc_sh2d.at[idx_v.at[j]], add=True)
        return 0
    acc_sh2d = acc_sh.reshape(N, 4)
    lax.fori_loop(0, NSUB, _scat, 0)

    plsc.subcore_barrier()

    @pl.when(sid < NS - 1)
    def _():
        pltpu.sync_copy(acc_sh.at[pl.ds(zbase * 4, ZROWS * 4)],
                        out_hbm.at[cid, pl.ds(zbase * 4, ZROWS * 4)])

    @pl.when(sid == NS - 1)
    def _():
        pltpu.sync_copy(acc_sh.at[pl.ds(15 * ZROWS * 4, last * 4)],
                        out_hbm.at[cid, pl.ds(15 * ZROWS * 4, last * 4)])


def _scatter_stage(forces_e4, dst):
    val = forces_e4.reshape(NW, EPW, 4)
    idx4 = dst.astype(jnp.int32).reshape(NW, NSUB, SCHUNK)
    mesh = plsc.VectorSubcoreMesh(core_axis_name="c", subcore_axis_name="s")
    scat = pl.kernel(
        _scatter_body,
        out_type=jax.ShapeDtypeStruct((NC, N * 4), jnp.float32),
        mesh=mesh,
        scratch_types=[
            pltpu.VMEM((NSUB, SCHUNK), jnp.int32),
            pltpu.VMEM((EPW, 4), jnp.float32),
            pltpu.VMEM((ZROWS * 4,), jnp.float32),
            pltpu.VMEM_SHARED((N * 4,), jnp.float32),
        ],
    )
    return scat(val, idx4)


def _reduce_body(p_ref, out_ref):
    out_ref[...] = p_ref[0] + p_ref[1]


def _reduce_stage(partials):
    return pl.pallas_call(
        _reduce_body,
        out_shape=jax.ShapeDtypeStruct((N, 4), jnp.float32),
    )(partials.reshape(NC, N, 4))


@jax.jit
def kernel(force_features, edge_vectors, edge_index_dst, pos,
           W0, b0, W1, b1, W2, b2, W3, b3, W4, b4):
    ev4 = jnp.pad(edge_vectors, ((0, 0), (0, 1)))
    w4row = W4.reshape(1, HIDDEN)
    ballb4 = jnp.concatenate(
        [jnp.stack([b0, b1, b2, b3]),
         jnp.broadcast_to(b4.reshape(1, 1), (1, HIDDEN)),
         jnp.zeros((3, HIDDEN), jnp.float32)], axis=0)
    forces_e4 = _mlp_stage(force_features, ev4, W0, W1, W2, W3, w4row, ballb4)
    partials = _scatter_stage(forces_e4, edge_index_dst)
    forces4 = _reduce_stage(partials)
    return forces4[:, :3]


# speedup vs baseline: 1.7621x; 1.7621x over previous
import functools

import jax
import jax.numpy as jnp
from jax import lax
from jax.experimental import pallas as pl
from jax.experimental.pallas import tpu as pltpu
from jax.experimental.pallas import tpu_sc as plsc

E = 160000
N = 10000
HIDDEN = 256


BLK_E = 640


def _silu(x):
    return x * (1.0 / (1.0 + jnp.exp(-x)))


def _mlp_body(ff_ref, ev_ref, w0_ref, w1_ref, w2_ref, w3_ref, w4_ref,
              b_ref, out_ref):
    h = ff_ref[...]
    h = _silu(jax.lax.dot_general(h, w0_ref[...], (((1,), (0,)), ((), ())),
                                  preferred_element_type=jnp.float32)
              + b_ref[0, :])
    h = _silu(jax.lax.dot_general(h, w1_ref[...], (((1,), (0,)), ((), ())),
                                  preferred_element_type=jnp.float32)
              + b_ref[1, :])
    h = _silu(jax.lax.dot_general(h, w2_ref[...], (((1,), (0,)), ((), ())),
                                  preferred_element_type=jnp.float32)
              + b_ref[2, :])
    h = _silu(jax.lax.dot_general(h, w3_ref[...], (((1,), (0,)), ((), ())),
                                  preferred_element_type=jnp.float32)
              + b_ref[3, :])
    scale = jnp.sum(h * w4_ref[...], axis=1, keepdims=True) + b_ref[4, 0]
    out_ref[...] = scale * ev_ref[...]


def _mlp_stage(ff, ev4, W0, W1, W2, W3, w4row, ballb4):
    grid = (E // BLK_E,)
    return pl.pallas_call(
        _mlp_body,
        grid=grid,
        in_specs=[
            pl.BlockSpec((BLK_E, HIDDEN), lambda i: (i, 0)),
            pl.BlockSpec((BLK_E, 8), lambda i: (i, 0)),
            pl.BlockSpec((HIDDEN, HIDDEN), lambda i: (0, 0)),
            pl.BlockSpec((HIDDEN, HIDDEN), lambda i: (0, 0)),
            pl.BlockSpec((HIDDEN, HIDDEN), lambda i: (0, 0)),
            pl.BlockSpec((HIDDEN, HIDDEN), lambda i: (0, 0)),
            pl.BlockSpec((1, HIDDEN), lambda i: (0, 0)),
            pl.BlockSpec((8, HIDDEN), lambda i: (0, 0)),
        ],
        out_specs=pl.BlockSpec((BLK_E, 8), lambda i: (i, 0)),
        out_shape=jax.ShapeDtypeStruct((E, 8), jnp.float32),
    )(ff, ev4, W0, W1, W2, W3, w4row, ballb4)


NC, NS = 2, 16
NW = NC * NS
EPW = E // NW
SCHUNK = 100
NSUB = EPW // SCHUNK
ZROWS = 640


def _scatter_body(val_hbm, idx_hbm, zero_hbm, out_hbm, idx_v, val_v, acc_sh):
    cid = lax.axis_index("c")
    sid = lax.axis_index("s")
    wid = sid * NC + cid

    zbase = sid * ZROWS
    last = N - (NS - 1) * ZROWS

    @pl.when(sid < NS - 1)
    def _():
        pltpu.sync_copy(zero_hbm.at[pl.ds(zbase, ZROWS)],
                        acc_sh.at[pl.ds(zbase, ZROWS)])

    @pl.when(sid == NS - 1)
    def _():
        pltpu.sync_copy(zero_hbm.at[pl.ds((NS - 1) * ZROWS, last)],
                        acc_sh.at[pl.ds((NS - 1) * ZROWS, last)])

    pltpu.sync_copy(idx_hbm.at[wid], idx_v)
    pltpu.sync_copy(val_hbm.at[wid], val_v)

    plsc.subcore_barrier()

    def _scat(j, _):
        pltpu.sync_copy(val_v.at[pl.ds(j * SCHUNK, SCHUNK)],
                        acc_sh.at[idx_v.at[j]], add=True)
        return 0
    lax.fori_loop(0, NSUB, _scat, 0)

    plsc.subcore_barrier()

    @pl.when(sid < NS - 1)
    def _():
        pltpu.sync_copy(acc_sh.at[pl.ds(zbase, ZROWS)],
                        out_hbm.at[cid, pl.ds(zbase, ZROWS)])

    @pl.when(sid == NS - 1)
    def _():
        pltpu.sync_copy(acc_sh.at[pl.ds((NS - 1) * ZROWS, last)],
                        out_hbm.at[cid, pl.ds((NS - 1) * ZROWS, last)])


def _scatter_stage(forces_e8, dst):
    val = forces_e8.reshape(NW, EPW, 8)
    idx4 = dst.astype(jnp.int32).reshape(NW, NSUB, SCHUNK)
    zeros = jnp.zeros((N, 8), jnp.float32)
    mesh = plsc.VectorSubcoreMesh(core_axis_name="c", subcore_axis_name="s")
    scat = pl.kernel(
        _scatter_body,
        out_type=jax.ShapeDtypeStruct((NC, N, 8), jnp.float32),
        mesh=mesh,
        compiler_params=pltpu.CompilerParams(use_tc_tiling_on_sc=False),
        scratch_types=[
            pltpu.VMEM((NSUB, SCHUNK), jnp.int32),
            pltpu.VMEM((EPW, 8), jnp.float32),
            pltpu.VMEM_SHARED((N, 8), jnp.float32),
        ],
    )
    return scat(val, idx4, zeros)


def _reduce_body(p_ref, out_ref):
    out_ref[...] = p_ref[0] + p_ref[1]


def _reduce_stage(partials):
    return pl.pallas_call(
        _reduce_body,
        out_shape=jax.ShapeDtypeStruct((N, 8), jnp.float32),
    )(partials)


@jax.jit
def kernel(force_features, edge_vectors, edge_index_dst, pos,
           W0, b0, W1, b1, W2, b2, W3, b3, W4, b4):
    ev8 = jnp.pad(edge_vectors, ((0, 0), (0, 5)))
    w4row = W4.reshape(1, HIDDEN)
    ballb4 = jnp.concatenate(
        [jnp.stack([b0, b1, b2, b3]),
         jnp.broadcast_to(b4.reshape(1, 1), (1, HIDDEN)),
         jnp.zeros((3, HIDDEN), jnp.float32)], axis=0)
    forces_e8 = _mlp_stage(force_features, ev8, W0, W1, W2, W3, w4row, ballb4)
    partials = _scatter_stage(forces_e8, edge_index_dst)
    forces4 = _reduce_stage(partials)
    return forces4[:, :3]
